# baseline (speedup 1.0000x reference)
import functools

import jax
import jax.numpy as jnp
from jax import lax
from jax.experimental import pallas as pl
from jax.experimental.pallas import tpu as pltpu
from jax.experimental.pallas import tpu_sc as plsc

NN = 10000
NP = 10240
EE = 320000
CH = 128
EP = 327680
NROW = EP // CH
NC, NS = 2, 16
RPT = NROW // (NC * NS)
RPTF = NROW // NS
RPS = NP // NS
HP = 80
BR = 1024
NB = 4

_mesh = plsc.VectorSubcoreMesh(core_axis_name="c", subcore_axis_name="s")
_sc_params = pltpu.CompilerParams(use_tc_tiling_on_sc=False)


def _zero_vmem_1d(ref, n):
    z = jnp.zeros((16,), jnp.float32)

    def body(i, _):
        for k in range(4):
            ref[pl.ds(i * 64 + k * 16, 16)] = z
        return 0
    lax.fori_loop(0, n // 64, body, 0)


def _propagate(tab, acc, src_v, dst_v, bufs, gsems, ssems, nrows):
    for k in range(NB):
        pltpu.async_copy(tab.at[src_v.at[k]], bufs[k], gsems[k])

    def body(j, _):
        for k in range(NB):
            r = j * NB + k
            pltpu.make_async_copy(tab.at[src_v.at[r]], bufs[k], gsems[k]).wait()
            pltpu.async_copy(bufs[k], acc.at[dst_v.at[r]], ssems[k], add=True)
        for k in range(NB):
            r = j * NB + k

            @pl.when(r + NB < nrows)
            def _():
                pltpu.make_async_copy(
                    bufs[k], acc.at[dst_v.at[r]], ssems[k]).wait()
                pltpu.async_copy(tab.at[src_v.at[r + NB]], bufs[k], gsems[k])
        return 0
    lax.fori_loop(0, nrows // NB, body, 0)
    for k in range(NB):
        r = nrows - NB + k
        pltpu.make_async_copy(bufs[k], acc.at[dst_v.at[r]], ssems[k]).wait()


EFLAT = RPTF * CH


def _reg_propagate(tab_v, acc_v, src_v, dst_v, n):
    def body(i, _):
        for k in range(4):
            ix = pl.ds(i * 64 + k * 16, 16)
            vals = plsc.load_gather(tab_v, [src_v[ix]])
            plsc.addupdate_scatter(acc_v, [dst_v[ix]], vals)
        return 0
    lax.fori_loop(0, n // 64, body, 0)


def _slot_reduce(slots, red_v, tmp_v, sl):
    _zero_vmem_1d(red_v, RPS)
    for k in range(NS):
        pltpu.sync_copy(slots.at[k, sl], tmp_v)

        def add_body(i, _):
            for j in range(4):
                ix = pl.ds(i * 64 + j * 16, 16)
                red_v[ix] = red_v[ix] + tmp_v[ix]
            return 0
        lax.fori_loop(0, RPS // 64, add_body, 0)


ECT = EP // (NC * NS)


@functools.partial(
    pl.kernel,
    out_type=jax.ShapeDtypeStruct((NC, NP), jnp.float32),
    mesh=_mesh,
    compiler_params=pltpu.CompilerParams(use_tc_tiling_on_sc=False,
                                         needs_layout_passes=False),
    scratch_types=[
        pltpu.VMEM((ECT,), jnp.int32),
        pltpu.VMEM((NP,), jnp.float32),
        pltpu.VMEM((RPS,), jnp.float32),
        pltpu.VMEM((RPS,), jnp.float32),
        pltpu.VMEM_SHARED((NS, NP), jnp.float32),
    ],
)
def _sc_deg(dst_hbm, out_hbm, dst_v, acc_v, red_v, tmp_v, slots):
    c = lax.axis_index("c")
    s = lax.axis_index("s")
    t = c * NS + s
    sl = pl.ds(s * RPS, RPS)
    pltpu.sync_copy(dst_hbm.at[pl.ds(t * ECT, ECT)], dst_v)
    _zero_vmem_1d(acc_v, NP)
    ones = jnp.ones((16,), jnp.float32)

    def body(i, _):
        for k in range(4):
            ix = pl.ds(i * 64 + k * 16, 16)
            plsc.addupdate_scatter(acc_v, [dst_v[ix]], ones)
        return 0
    lax.fori_loop(0, ECT // 64, body, 0)

    pltpu.sync_copy(acc_v, slots.at[s])
    plsc.subcore_barrier()
    _slot_reduce(slots, red_v, tmp_v, sl)
    pltpu.sync_copy(red_v, out_hbm.at[c, sl])


@functools.partial(
    pl.kernel,
    out_type=jax.ShapeDtypeStruct((NC, NP, HP), jnp.float32),
    mesh=_mesh,
    compiler_params=_sc_params,
    scratch_types=[
        pltpu.VMEM((RPT, CH), jnp.int32),
        pltpu.VMEM((RPT, CH), jnp.int32),
        [pltpu.VMEM((CH, HP), jnp.float32)] * NB,
        pltpu.VMEM((CH, HP), jnp.float32),
        pltpu.VMEM_SHARED((NP, HP), jnp.float32),
        [pltpu.SemaphoreType.DMA] * NB,
        [pltpu.SemaphoreType.DMA] * NB,
    ],
)
def _sc_wide(tab_hbm, src_hbm, dst_hbm, out_hbm,
             src_v, dst_v, bufs, zbuf, acc, gsems, ssems):
    c = lax.axis_index("c")
    s = lax.axis_index("s")
    t = c * NS + s
    pltpu.sync_copy(src_hbm.at[pl.ds(t * RPT, RPT)], src_v)
    pltpu.sync_copy(dst_hbm.at[pl.ds(t * RPT, RPT)], dst_v)

    def zrow(i, _):
        def zcol(k, _):
            zbuf[i, pl.ds(k * 16, 16)] = jnp.zeros((16,), jnp.float32)
            return 0
        lax.fori_loop(0, HP // 16, zcol, 0)
        return 0
    lax.fori_loop(0, CH, zrow, 0)
    for k in range(RPS // CH):
        pltpu.sync_copy(zbuf, acc.at[pl.ds(s * RPS + k * CH, CH)])
    plsc.subcore_barrier()

    _propagate(tab_hbm, acc, src_v, dst_v, bufs, gsems, ssems, RPT)

    plsc.subcore_barrier()
    for k in range(RPS // CH):
        pltpu.sync_copy(acc.at[pl.ds(s * RPS + k * CH, CH)],
                        out_hbm.at[c, pl.ds(s * RPS + k * CH, CH)])


@functools.partial(
    pl.kernel,
    out_type=jax.ShapeDtypeStruct((NP,), jnp.float32),
    mesh=_mesh,
    compiler_params=pltpu.CompilerParams(use_tc_tiling_on_sc=False,
                                         needs_layout_passes=False),
    scratch_types=[
        pltpu.VMEM((EFLAT,), jnp.int32),
        pltpu.VMEM((EFLAT,), jnp.int32),
        pltpu.VMEM((NP,), jnp.float32),
        pltpu.VMEM((NP,), jnp.float32),
        pltpu.VMEM((RPS,), jnp.float32),
        pltpu.VMEM((RPS,), jnp.float32),
        pltpu.VMEM((RPS,), jnp.float32),
        pltpu.VMEM((RPS,), jnp.float32),
        pltpu.VMEM((16,), jnp.float32),
        pltpu.VMEM((16,), jnp.float32),
        pltpu.VMEM_SHARED((NS, NP), jnp.float32),
        pltpu.VMEM_SHARED((NP,), jnp.float32),
    ],
)
def _sc_final(ut_hbm, dinv_hbm, cvec_hbm, b3vec_hbm, src_hbm, dst_hbm,
              out_hbm, src_v, dst_v, tab_v, acc_v, dinv_v, red_v, tmp_v,
              vt_v, c_v, b3_v, slots, tab_s):
    c = lax.axis_index("c")
    s = lax.axis_index("s")
    sl = pl.ds(s * RPS, RPS)
    pltpu.sync_copy(src_hbm.at[pl.ds(s * EFLAT, EFLAT)], src_v)
    pltpu.sync_copy(dst_hbm.at[pl.ds(s * EFLAT, EFLAT)], dst_v)
    pltpu.sync_copy(ut_hbm, tab_v)
    pltpu.sync_copy(dinv_hbm.at[sl], dinv_v)
    pltpu.sync_copy(cvec_hbm, c_v)
    pltpu.sync_copy(b3vec_hbm, b3_v)

    _zero_vmem_1d(acc_v, NP)
    _reg_propagate(tab_v, acc_v, src_v, dst_v, EFLAT)
    pltpu.sync_copy(acc_v, slots.at[s])
    plsc.subcore_barrier()
    _slot_reduce(slots, red_v, tmp_v, sl)

    cb = c_v[...]

    def vt_body(i, _):
        ix = pl.ds(i * 16, 16)
        dv = dinv_v[ix]
        vt_v[ix] = dv * (dv * (red_v[ix] + tab_v[pl.ds(s * RPS + i * 16, 16)]) + cb)
        return 0
    lax.fori_loop(0, RPS // 16, vt_body, 0)
    pltpu.sync_copy(vt_v, tab_s.at[sl])
    plsc.subcore_barrier()
    pltpu.sync_copy(tab_s, tab_v)

    _zero_vmem_1d(acc_v, NP)
    _reg_propagate(tab_v, acc_v, src_v, dst_v, EFLAT)
    pltpu.sync_copy(acc_v, slots.at[s])
    plsc.subcore_barrier()
    _slot_reduce(slots, red_v, tmp_v, sl)

    bb = b3_v[...]

    def out_body(i, _):
        ix = pl.ds(i * 16, 16)
        vt_v[ix] = dinv_v[ix] * (red_v[ix] + vt_v[ix]) + bb
        return 0
    lax.fori_loop(0, RPS // 16, out_body, 0)

    @pl.when(c == 0)
    def _():
        pltpu.sync_copy(vt_v, out_hbm.at[sl])


def _tc_mm_body(x_ref, w_ref, t1_ref):
    t1_ref[...] = jnp.dot(x_ref[...], w_ref[...],
                          preferred_element_type=jnp.float32)


def _tc_mm(xp, w1p):
    return pl.pallas_call(
        _tc_mm_body,
        grid=(NP // BR,),
        in_specs=[
            pl.BlockSpec((BR, 128), lambda i: (i, 0)),
            pl.BlockSpec((128, HP), lambda i: (0, 0)),
        ],
        out_specs=pl.BlockSpec((BR, HP), lambda i: (i, 0)),
        out_shape=jax.ShapeDtypeStruct((NP, HP), jnp.float32),
    )(xp, w1p)


def _tc_scale_body(t1_ref, degs_ref, t1s_ref, dinv_ref):
    deg = degs_ref[0] + degs_ref[1] + 1.0
    dinv = lax.rsqrt(deg)
    t1s_ref[...] = t1_ref[...] * dinv
    dinv_ref[...] = dinv


def _tc_scale(t1, degs):
    return pl.pallas_call(
        _tc_scale_body,
        grid=(NP // BR,),
        in_specs=[
            pl.BlockSpec((BR, HP), lambda i: (i, 0)),
            pl.BlockSpec((NC, BR, 1), lambda i: (0, i, 0)),
        ],
        out_specs=[
            pl.BlockSpec((BR, HP), lambda i: (i, 0)),
            pl.BlockSpec((BR, 1), lambda i: (i, 0)),
        ],
        out_shape=[
            jax.ShapeDtypeStruct((NP, HP), jnp.float32),
            jax.ShapeDtypeStruct((NP, 1), jnp.float32),
        ],
    )(t1, degs)


def _tc_b_body(parts_ref, t1s_ref, dinv_ref, b1_ref, w2_ref, w3_ref, b2_ref,
               ut_ref, misc_ref):
    dinv = dinv_ref[...]
    p1 = dinv * (parts_ref[0] + parts_ref[1] + t1s_ref[...])
    h1r = jnp.maximum(p1 + b1_ref[...], 0.0)
    u2 = jnp.dot(h1r, w2_ref[...], preferred_element_type=jnp.float32)
    u = jnp.dot(u2, w3_ref[...], preferred_element_type=jnp.float32)
    ut_ref[...] = dinv * u
    cterm = jnp.sum(b2_ref[...] * w3_ref[...].reshape(1, 96))
    misc_ref[...] = jnp.full((1, 16), cterm, jnp.float32)


def _tc_b(parts, t1s, dinv, b1p, w2p, w3p, b2p):
    return pl.pallas_call(
        _tc_b_body,
        grid=(NP // BR,),
        in_specs=[
            pl.BlockSpec((NC, BR, HP), lambda i: (0, i, 0)),
            pl.BlockSpec((BR, HP), lambda i: (i, 0)),
            pl.BlockSpec((BR, 1), lambda i: (i, 0)),
            pl.BlockSpec((1, HP), lambda i: (0, 0)),
            pl.BlockSpec((HP, 96), lambda i: (0, 0)),
            pl.BlockSpec((96, 1), lambda i: (0, 0)),
            pl.BlockSpec((1, 96), lambda i: (0, 0)),
        ],
        out_specs=[
            pl.BlockSpec((BR, 1), lambda i: (i, 0)),
            pl.BlockSpec((1, 16), lambda i: (0, 0)),
        ],
        out_shape=[
            jax.ShapeDtypeStruct((NP, 1), jnp.float32),
            jax.ShapeDtypeStruct((1, 16), jnp.float32),
        ],
    )(parts, t1s, dinv, b1p, w2p, w3p, b2p)


def kernel(x, edge_index, W1, b1, W2, b2, W3, b3):
    f32 = jnp.float32
    xp = jnp.zeros((NP, 128), f32).at[:NN].set(x)
    w1p = jnp.zeros((128, HP), f32).at[:, :71].set(W1)
    b1p = jnp.zeros((1, HP), f32).at[0, :71].set(b1)
    w2p = jnp.zeros((HP, 96), f32).at[:71, :82].set(W2)
    w3p = jnp.zeros((96, 1), f32).at[:82].set(W3)
    b2p = jnp.zeros((1, 96), f32).at[0, :82].set(b2)
    b3vec = jnp.broadcast_to(b3.astype(f32), (16,))

    npad = EP - EE
    spread = NN + (jnp.arange(npad, dtype=jnp.int32) % (NP - NN))
    src1d = jnp.concatenate([edge_index[0], spread])
    dst1d = jnp.concatenate([edge_index[1], spread])
    src2d = src1d.reshape(NROW, CH)
    dst2d = dst1d.reshape(NROW, CH)

    degs = _sc_deg(dst1d)
    t1 = _tc_mm(xp, w1p)
    t1s, dinv = _tc_scale(t1, degs.reshape(NC, NP, 1))
    parts = _sc_wide(t1s, src2d, dst2d)
    ut, misc = _tc_b(parts, t1s, dinv, b1p, w2p, w3p, b2p)
    out = _sc_final(ut.reshape(NP), dinv.reshape(NP), misc.reshape(16),
                    b3vec, src1d, dst1d)
    return out[:NN].reshape(NN, 1)


# speedup vs baseline: 1.0400x; 1.0400x over previous
import functools

import jax
import jax.numpy as jnp
from jax import lax
from jax.experimental import pallas as pl
from jax.experimental.pallas import tpu as pltpu
from jax.experimental.pallas import tpu_sc as plsc

NN = 10000
NP = 10240
EE = 320000
CH = 128
EP = 327680
NROW = EP // CH
NC, NS = 2, 16
RPT = NROW // (NC * NS)
RPTF = NROW // NS
RPS = NP // NS
HP = 80
BR = 1024
NB = 4

_mesh = plsc.VectorSubcoreMesh(core_axis_name="c", subcore_axis_name="s")
_sc_params = pltpu.CompilerParams(use_tc_tiling_on_sc=False)


def _zero_vmem_1d(ref, n):
    z = jnp.zeros((16,), jnp.float32)

    def body(i, _):
        for k in range(4):
            ref[pl.ds(i * 64 + k * 16, 16)] = z
        return 0
    lax.fori_loop(0, n // 64, body, 0)


def _propagate(tab, acc, src_v, dst_v, bufs, gsems, ssems, nrows):
    nb = len(bufs)
    for k in range(nb):
        pltpu.async_copy(tab.at[src_v.at[k]], bufs[k], gsems[k])

    def body(j, _):
        for k in range(nb):
            r = j * nb + k
            pltpu.make_async_copy(tab.at[src_v.at[r]], bufs[k], gsems[k]).wait()
            pltpu.async_copy(bufs[k], acc.at[dst_v.at[r]], ssems[k], add=True)
        for k in range(nb):
            r = j * nb + k

            @pl.when(r + nb < nrows)
            def _():
                pltpu.make_async_copy(
                    bufs[k], acc.at[dst_v.at[r]], ssems[k]).wait()
                pltpu.async_copy(tab.at[src_v.at[r + nb]], bufs[k], gsems[k])
        return 0
    lax.fori_loop(0, nrows // nb, body, 0)
    for k in range(nb):
        r = nrows - nb + k
        pltpu.make_async_copy(bufs[k], acc.at[dst_v.at[r]], ssems[k]).wait()


@functools.partial(
    pl.kernel,
    out_type=jax.ShapeDtypeStruct((NC, NP), jnp.float32),
    mesh=_mesh,
    compiler_params=_sc_params,
    scratch_types=[
        pltpu.VMEM((RPT, CH), jnp.int32),
        pltpu.VMEM((CH,), jnp.float32),
        pltpu.VMEM((RPS,), jnp.float32),
        pltpu.VMEM_SHARED((NP,), jnp.float32),
        pltpu.SemaphoreType.DMA,
    ],
)
def _sc_deg(dst_hbm, out_hbm, dst_v, ones_v, zero_v, acc, sem):
    c = lax.axis_index("c")
    s = lax.axis_index("s")
    t = c * NS + s
    pltpu.sync_copy(dst_hbm.at[pl.ds(t * RPT, RPT)], dst_v)

    def ones_body(i, _):
        ones_v[pl.ds(i * 16, 16)] = jnp.ones((16,), jnp.float32)
        return 0
    lax.fori_loop(0, CH // 16, ones_body, 0)

    _zero_vmem_1d(zero_v, RPS)
    pltpu.sync_copy(zero_v, acc.at[pl.ds(s * RPS, RPS)])
    plsc.subcore_barrier()

    def body(r, _):
        pltpu.async_copy(ones_v, acc.at[dst_v.at[r]], sem, add=True)
        return 0
    lax.fori_loop(0, RPT, body, 0)

    def drain(r, _):
        pltpu.make_async_copy(ones_v, acc.at[dst_v.at[r]], sem).wait()
        return 0
    lax.fori_loop(0, RPT, drain, 0)

    plsc.subcore_barrier()
    pltpu.sync_copy(acc.at[pl.ds(s * RPS, RPS)],
                    out_hbm.at[c, pl.ds(s * RPS, RPS)])


@functools.partial(
    pl.kernel,
    out_type=jax.ShapeDtypeStruct((NC, NP, HP), jnp.float32),
    mesh=_mesh,
    compiler_params=_sc_params,
    scratch_types=[
        pltpu.VMEM((RPT, CH), jnp.int32),
        pltpu.VMEM((RPT, CH), jnp.int32),
        [pltpu.VMEM((CH, HP), jnp.float32)] * NB,
        pltpu.VMEM((CH, HP), jnp.float32),
        pltpu.VMEM_SHARED((NP, HP), jnp.float32),
        [pltpu.SemaphoreType.DMA] * NB,
        [pltpu.SemaphoreType.DMA] * NB,
    ],
)
def _sc_wide(tab_hbm, src_hbm, dst_hbm, out_hbm,
             src_v, dst_v, bufs, zbuf, acc, gsems, ssems):
    c = lax.axis_index("c")
    s = lax.axis_index("s")
    t = c * NS + s
    pltpu.sync_copy(src_hbm.at[pl.ds(t * RPT, RPT)], src_v)
    pltpu.sync_copy(dst_hbm.at[pl.ds(t * RPT, RPT)], dst_v)

    def zrow(i, _):
        def zcol(k, _):
            zbuf[i, pl.ds(k * 16, 16)] = jnp.zeros((16,), jnp.float32)
            return 0
        lax.fori_loop(0, HP // 16, zcol, 0)
        return 0
    lax.fori_loop(0, CH, zrow, 0)
    for k in range(RPS // CH):
        pltpu.sync_copy(zbuf, acc.at[pl.ds(s * RPS + k * CH, CH)])
    plsc.subcore_barrier()

    _propagate(tab_hbm, acc, src_v, dst_v, bufs, gsems, ssems, RPT)

    plsc.subcore_barrier()
    for k in range(RPS // CH):
        pltpu.sync_copy(acc.at[pl.ds(s * RPS + k * CH, CH)],
                        out_hbm.at[c, pl.ds(s * RPS + k * CH, CH)])


NBS = 8


@functools.partial(
    pl.kernel,
    out_type=jax.ShapeDtypeStruct((NP,), jnp.float32),
    mesh=_mesh,
    compiler_params=_sc_params,
    scratch_types=[
        pltpu.VMEM((RPTF, CH), jnp.int32),
        pltpu.VMEM((RPTF, CH), jnp.int32),
        [pltpu.VMEM((CH,), jnp.float32)] * NBS,
        pltpu.VMEM((RPS,), jnp.float32),
        pltpu.VMEM((RPS,), jnp.float32),
        pltpu.VMEM((RPS,), jnp.float32),
        pltpu.VMEM((RPS,), jnp.float32),
        pltpu.VMEM((RPS,), jnp.float32),
        pltpu.VMEM((16,), jnp.float32),
        pltpu.VMEM((16,), jnp.float32),
        pltpu.VMEM_SHARED((NP,), jnp.float32),
        pltpu.VMEM_SHARED((NP,), jnp.float32),
        [pltpu.SemaphoreType.DMA] * NBS,
        [pltpu.SemaphoreType.DMA] * NBS,
    ],
)
def _sc_final(ut_hbm, dinv_hbm, cvec_hbm, b3vec_hbm, src_hbm, dst_hbm,
              out_hbm, src_v, dst_v, bufs, zero_v, ut_v, dinv_v, su_v,
              vt_v, c_v, b3_v, tab_s, acc_s, gsems, ssems):
    c = lax.axis_index("c")
    s = lax.axis_index("s")
    sl = pl.ds(s * RPS, RPS)
    pltpu.sync_copy(src_hbm.at[pl.ds(s * RPTF, RPTF)], src_v)
    pltpu.sync_copy(dst_hbm.at[pl.ds(s * RPTF, RPTF)], dst_v)
    pltpu.sync_copy(ut_hbm.at[sl], ut_v)
    pltpu.sync_copy(dinv_hbm.at[sl], dinv_v)
    pltpu.sync_copy(cvec_hbm, c_v)
    pltpu.sync_copy(b3vec_hbm, b3_v)
    _zero_vmem_1d(zero_v, RPS)
    pltpu.sync_copy(zero_v, acc_s.at[sl])
    pltpu.sync_copy(ut_v, tab_s.at[sl])
    plsc.subcore_barrier()

    _propagate(tab_s, acc_s, src_v, dst_v, bufs, gsems, ssems, RPTF)
    plsc.subcore_barrier()

    pltpu.sync_copy(acc_s.at[sl], su_v)
    cb = c_v[...]

    def vt_body(i, _):
        ix = pl.ds(i * 16, 16)
        dv = dinv_v[ix]
        vt_v[ix] = dv * (dv * (su_v[ix] + ut_v[ix]) + cb)
        return 0
    lax.fori_loop(0, RPS // 16, vt_body, 0)
    pltpu.sync_copy(zero_v, acc_s.at[sl])
    pltpu.sync_copy(vt_v, tab_s.at[sl])
    plsc.subcore_barrier()

    _propagate(tab_s, acc_s, src_v, dst_v, bufs, gsems, ssems, RPTF)
    plsc.subcore_barrier()

    pltpu.sync_copy(acc_s.at[sl], su_v)
    bb = b3_v[...]

    def out_body(i, _):
        ix = pl.ds(i * 16, 16)
        ut_v[ix] = dinv_v[ix] * (su_v[ix] + vt_v[ix]) + bb
        return 0
    lax.fori_loop(0, RPS // 16, out_body, 0)

    @pl.when(c == 0)
    def _():
        pltpu.sync_copy(ut_v, out_hbm.at[sl])


def _tc_a_body(x_ref, w_ref, degs_ref, t1s_ref, dinv_ref):
    deg = degs_ref[0] + degs_ref[1] + 1.0
    dinv = lax.rsqrt(deg)
    t1 = jnp.dot(x_ref[...], w_ref[...], preferred_element_type=jnp.float32)
    t1s_ref[...] = t1 * dinv
    dinv_ref[...] = dinv


def _tc_a(xp, w1p, degs):
    return pl.pallas_call(
        _tc_a_body,
        grid=(NP // BR,),
        in_specs=[
            pl.BlockSpec((BR, 128), lambda i: (i, 0)),
            pl.BlockSpec((128, HP), lambda i: (0, 0)),
            pl.BlockSpec((NC, BR, 1), lambda i: (0, i, 0)),
        ],
        out_specs=[
            pl.BlockSpec((BR, HP), lambda i: (i, 0)),
            pl.BlockSpec((BR, 1), lambda i: (i, 0)),
        ],
        out_shape=[
            jax.ShapeDtypeStruct((NP, HP), jnp.float32),
            jax.ShapeDtypeStruct((NP, 1), jnp.float32),
        ],
    )(xp, w1p, degs)


def _tc_b_body(parts_ref, t1s_ref, dinv_ref, b1_ref, w2_ref, w3_ref, b2_ref,
               ut_ref, misc_ref):
    dinv = dinv_ref[...]
    p1 = dinv * (parts_ref[0] + parts_ref[1] + t1s_ref[...])
    h1r = jnp.maximum(p1 + b1_ref[...], 0.0)
    u2 = jnp.dot(h1r, w2_ref[...], preferred_element_type=jnp.float32)
    u = jnp.dot(u2, w3_ref[...], preferred_element_type=jnp.float32)
    ut_ref[...] = dinv * u
    cterm = jnp.sum(b2_ref[...] * w3_ref[...].reshape(1, 96))
    misc_ref[...] = jnp.full((1, 16), cterm, jnp.float32)


def _tc_b(parts, t1s, dinv, b1p, w2p, w3p, b2p):
    return pl.pallas_call(
        _tc_b_body,
        grid=(NP // BR,),
        in_specs=[
            pl.BlockSpec((NC, BR, HP), lambda i: (0, i, 0)),
            pl.BlockSpec((BR, HP), lambda i: (i, 0)),
            pl.BlockSpec((BR, 1), lambda i: (i, 0)),
            pl.BlockSpec((1, HP), lambda i: (0, 0)),
            pl.BlockSpec((HP, 96), lambda i: (0, 0)),
            pl.BlockSpec((96, 1), lambda i: (0, 0)),
            pl.BlockSpec((1, 96), lambda i: (0, 0)),
        ],
        out_specs=[
            pl.BlockSpec((BR, 1), lambda i: (i, 0)),
            pl.BlockSpec((1, 16), lambda i: (0, 0)),
        ],
        out_shape=[
            jax.ShapeDtypeStruct((NP, 1), jnp.float32),
            jax.ShapeDtypeStruct((1, 16), jnp.float32),
        ],
    )(parts, t1s, dinv, b1p, w2p, w3p, b2p)


def kernel(x, edge_index, W1, b1, W2, b2, W3, b3):
    f32 = jnp.float32
    xp = jnp.zeros((NP, 128), f32).at[:NN].set(x)
    w1p = jnp.zeros((128, HP), f32).at[:, :71].set(W1)
    b1p = jnp.zeros((1, HP), f32).at[0, :71].set(b1)
    w2p = jnp.zeros((HP, 96), f32).at[:71, :82].set(W2)
    w3p = jnp.zeros((96, 1), f32).at[:82].set(W3)
    b2p = jnp.zeros((1, 96), f32).at[0, :82].set(b2)
    b3vec = jnp.broadcast_to(b3.astype(f32), (16,))

    npad = EP - EE
    spread = NN + (jnp.arange(npad, dtype=jnp.int32) % (NP - NN))
    src1d = jnp.concatenate([edge_index[0], spread])
    dst1d = jnp.concatenate([edge_index[1], spread])
    src2d = src1d.reshape(NROW, CH)
    dst2d = dst1d.reshape(NROW, CH)

    degs = _sc_deg(dst2d)
    t1s, dinv = _tc_a(xp, w1p, degs.reshape(NC, NP, 1))
    parts = _sc_wide(t1s, src2d, dst2d)
    ut, misc = _tc_b(parts, t1s, dinv, b1p, w2p, w3p, b2p)
    out = _sc_final(ut.reshape(NP), dinv.reshape(NP), misc.reshape(16),
                    b3vec, src2d, dst2d)
    return out[:NN].reshape(NN, 1)
